# baseline (speedup 1.0000x reference)
import functools

import jax
import jax.numpy as jnp
import numpy as np
from jax import lax
from jax.experimental import pallas as pl
from jax.experimental.pallas import tpu as pltpu
from jax.experimental.pallas import tpu_sc as plsc
from jax._src.pallas import primitives as _pl_primitives

_T = 1280
_BIG = 1e30
_NC, _NS, _L = 2, 16, 16
_NW = _NC * _NS


_CH = 8


def _pass1_tile(rowt_ref, col_ref, mask_ref, o2_ref, masked):
    col = col_ref[0]
    cx1 = col[0:1, :]
    cy1 = col[1:2, :]
    cx2 = col[2:3, :]
    cy2 = col[3:4, :]
    ca = col[4:5, :]
    acc = None
    for ci in range(_T // _CH):
        r0 = ci * _CH
        rt = rowt_ref[0, r0:r0 + _CH, :]
        rx1 = rt[:, 0:1]
        ry1 = rt[:, 1:2]
        rx2 = rt[:, 2:3]
        ry2 = rt[:, 3:4]
        ra = rt[:, 4:5]
        iw = jnp.maximum(jnp.minimum(rx2, cx2) - jnp.maximum(rx1, cx1), 0.0)
        ih = jnp.maximum(jnp.minimum(ry2, cy2) - jnp.maximum(ry1, cy1), 0.0)
        inter = iw * ih
        union = ra + ca - inter
        iou = inter * _pl_primitives.reciprocal(union, approx=True)
        v = iou * iou
        if masked:
            v = v * mask_ref[r0:r0 + _CH, :]
        o2_ref[0, 0, r0:r0 + _CH, :] = v
        acc = v if acc is None else jnp.maximum(acc, v)
    return jnp.max(acc, axis=0, keepdims=True)


def _pass1_body(ik_ref, jk_ref, rowt_ref, col_ref, mask_ref,
                o_ref, o2_ref, ot_ref, omin_ref, *, n_real):
    k = pl.program_id(1)
    i = ik_ref[k]
    j = jk_ref[k]

    @pl.when(k == 0)
    def _init_min():
        omin_ref[...] = jnp.full_like(omin_ref, _BIG)

    @pl.when(i == 0)
    def _init():
        o_ref[...] = jnp.zeros_like(o_ref)

    def accum(masked):
        m = _pass1_tile(rowt_ref, col_ref, mask_ref, o2_ref, masked)
        o_ref[0] = jnp.maximum(o_ref[0], jnp.broadcast_to(m, o_ref.shape[1:]))

    @pl.when(i < j)
    def _upper():
        accum(False)

    @pl.when(i == j)
    def _diag():
        accum(True)
        mx2 = o_ref[0, 0:1, :]
        gj_lane = j * _T + lax.broadcasted_iota(jnp.int32, (1, _T), 1)
        mx2 = jnp.where(gj_lane < n_real, mx2, _BIG)
        lane = lax.broadcasted_iota(jnp.int32, omin_ref.shape[1:], 1)
        omin_ref[0] = jnp.where(lane == j, jnp.min(mx2), omin_ref[0])
        ot_ref[0] = (jnp.broadcast_to(mx2, o_ref.shape[1:])).T


def _pass2_body(ik_ref, jk_ref, t2_ref, mxt_ref, mn_ref, o_ref):
    k = pl.program_id(1)
    i = ik_ref[k]
    j = jk_ref[k]

    @pl.when(i == 0)
    def _init():
        lane = lax.broadcasted_iota(jnp.int32, mn_ref.shape[1:], 1)
        s = -jnp.min(jnp.where(lane > j, mn_ref[0], _BIG))
        o_ref[...] = jnp.full_like(o_ref, 0.0) + s

    mx2c = mxt_ref[0][:, 0:1]
    acc = None
    for ci in range(_T // _CH):
        r0 = ci * _CH
        v = t2_ref[0, 0, r0:r0 + _CH, :] - mx2c[r0:r0 + _CH, :]
        acc = v if acc is None else jnp.maximum(acc, v)
    m = jnp.max(acc, axis=0, keepdims=True)
    o_ref[0] = jnp.maximum(o_ref[0], jnp.broadcast_to(m, o_ref.shape[1:]))


def _run_passes(coords, coords_t, n_real):
    b, _, npad = coords.shape
    nt = npad // _T
    pairs = [(i, j) for j in range(nt) for i in range(j + 1)]
    ik = jnp.asarray(np.array([p[0] for p in pairs], np.int32))
    jk = jnp.asarray(np.array([p[1] for p in pairs], np.int32))
    nk = len(pairs)
    cparams = pltpu.CompilerParams(
        dimension_semantics=("parallel", "arbitrary"))

    mask = jnp.asarray(np.triu(np.ones((_T, _T), np.float32), 1))

    rowt_spec = pl.BlockSpec((1, _T, 8), lambda b_, k_, ik_, jk_: (b_, ik_[k_], 0))
    col_spec = pl.BlockSpec((1, 8, _T), lambda b_, k_, ik_, jk_: (b_, 0, jk_[k_]))
    mask_spec = pl.BlockSpec((_T, _T), lambda b_, k_, ik_, jk_: (0, 0))
    out_spec = pl.BlockSpec((1, 8, _T), lambda b_, k_, ik_, jk_: (b_, 0, jk_[k_]))
    outt_spec = pl.BlockSpec((1, _T, 8), lambda b_, k_, ik_, jk_: (b_, jk_[k_], 0))
    min_spec = pl.BlockSpec((1, 8, 128), lambda b_, k_, ik_, jk_: (b_, 0, 0))

    tile_spec = pl.BlockSpec((1, 1, _T, _T), lambda b_, k_, ik_, jk_: (b_, k_, 0, 0))

    ioumax, iou2_tiles, mx2_t, tile_min2 = pl.pallas_call(
        functools.partial(_pass1_body, n_real=n_real),
        grid_spec=pltpu.PrefetchScalarGridSpec(
            num_scalar_prefetch=2,
            grid=(b, nk),
            in_specs=[rowt_spec, col_spec, mask_spec],
            out_specs=[out_spec, tile_spec, outt_spec, min_spec],
        ),
        out_shape=[jax.ShapeDtypeStruct((b, 8, npad), jnp.float32),
                   jax.ShapeDtypeStruct((b, nk, _T, _T), jnp.float32),
                   jax.ShapeDtypeStruct((b, npad, 8), jnp.float32),
                   jax.ShapeDtypeStruct((b, 8, 128), jnp.float32)],
        compiler_params=cparams,
    )(ik, jk, coords_t, coords, mask)
    del ioumax

    m = pl.pallas_call(
        _pass2_body,
        grid_spec=pltpu.PrefetchScalarGridSpec(
            num_scalar_prefetch=2,
            grid=(b, nk),
            in_specs=[tile_spec, rowt_spec, min_spec],
            out_specs=out_spec,
        ),
        out_shape=jax.ShapeDtypeStruct((b, 8, npad), jnp.float32),
        compiler_params=cparams,
    )(ik, jk, iou2_tiles, mx2_t, tile_min2)
    return m


def _sc_sort_gather(boxes_pad, si_pad):
    bsz, _, npad = boxes_pad.shape
    wpb = _NW // bsz
    chunk = npad // wpb
    nsteps = chunk // _L
    mesh = plsc.VectorSubcoreMesh(core_axis_name="c", subcore_axis_name="s")

    @functools.partial(
        pl.kernel, mesh=mesh,
        out_type=[jax.ShapeDtypeStruct((bsz, 8, npad), jnp.float32),
                  jax.ShapeDtypeStruct((bsz, npad * 8), jnp.float32)],
        scratch_types=[
            pltpu.VMEM((4 * npad,), jnp.float32),
            pltpu.VMEM((chunk,), jnp.int32),
            pltpu.VMEM((8, chunk), jnp.float32),
            pltpu.VMEM((chunk * 8,), jnp.float32),
        ],
        compiler_params=pltpu.CompilerParams(needs_layout_passes=False),
    )
    def k(boxes_hbm, si_hbm, out_hbm, outt_hbm, box_v, idx_v, o_v, ot_v):
        wid = lax.axis_index("s") * _NC + lax.axis_index("c")
        b = wid // wpb
        base = (wid % wpb) * chunk
        for c in range(4):
            pltpu.sync_copy(boxes_hbm.at[b, c],
                            box_v.at[pl.ds(c * npad, npad)])
        pltpu.sync_copy(si_hbm.at[b, pl.ds(base, chunk)], idx_v)
        zero = jnp.zeros((_L,), jnp.float32)
        lane = lax.iota(jnp.int32, _L)

        def body(s, _):
            off = s * _L
            idx = idx_v[pl.ds(off, _L)]
            cs = [plsc.load_gather(box_v, [idx + c * npad])
                  for c in range(4)]
            area = (jnp.maximum(cs[2] - cs[0], 0.0)
                    * jnp.maximum(cs[3] - cs[1], 0.0))
            tbase = (off + lane) * 8
            for c in range(4):
                o_v[c, pl.ds(off, _L)] = cs[c]
                plsc.store_scatter(ot_v, [tbase + c], cs[c])
            o_v[4, pl.ds(off, _L)] = area + 1e-12
            plsc.store_scatter(ot_v, [tbase + 4], area)
            for c in range(5, 8):
                o_v[c, pl.ds(off, _L)] = zero
            return _

        lax.fori_loop(0, nsteps, body, None)
        pltpu.sync_copy(o_v, out_hbm.at[b, pl.ds(0, 8), pl.ds(base, chunk)])
        pltpu.sync_copy(ot_v, outt_hbm.at[b, pl.ds(base * 8, chunk * 8)])

    return k(boxes_pad, si_pad)


def _sc_unsort_combine(m_pad, si_pad, mw_pad, ds_vec):
    bsz, _, npad = m_pad.shape
    n_masks = ds_vec.shape[0]
    wpb = _NW // bsz
    chunk = npad // wpb
    mesh = plsc.VectorSubcoreMesh(core_axis_name="c", subcore_axis_name="s")

    @functools.partial(
        pl.kernel, mesh=mesh,
        out_type=jax.ShapeDtypeStruct((bsz, npad), jnp.float32),
        scratch_types=[
            pltpu.VMEM((npad,), jnp.int32),
            pltpu.VMEM((npad,), jnp.int32),
            pltpu.VMEM((npad,), jnp.float32),
            pltpu.VMEM((n_masks, chunk), jnp.float32),
            pltpu.VMEM((n_masks, _L), jnp.float32),
            pltpu.VMEM((chunk,), jnp.float32),
        ],
        compiler_params=pltpu.CompilerParams(needs_layout_passes=False),
    )
    def k(m_hbm, si_hbm, mw_hbm, ds_hbm, out_hbm,
          si_v, un_v, m_v, w_v, ds_v, o_v):
        wid = lax.axis_index("s") * _NC + lax.axis_index("c")
        b = wid // wpb
        base = (wid % wpb) * chunk
        pltpu.sync_copy(si_hbm.at[b], si_v)
        pltpu.sync_copy(m_hbm.at[b, 0], m_v)
        pltpu.sync_copy(mw_hbm.at[b, pl.ds(0, n_masks), pl.ds(base, chunk)],
                        w_v)
        pltpu.sync_copy(ds_hbm, ds_v)
        lane = lax.iota(jnp.int32, _L)

        def scat(s, _):
            off = s * _L
            plsc.store_scatter(un_v, [si_v[pl.ds(off, _L)]], off + lane)
            return _

        lax.fori_loop(0, npad // _L, scat, None)

        def body(s, _):
            off = s * _L
            u = un_v[pl.ds(base + off, _L)]
            g = plsc.load_gather(m_v, [u])
            acc = jnp.zeros((_L,), jnp.float32)
            for mi in range(n_masks):
                acc = acc + jnp.exp(g * ds_v[mi]) * w_v[mi, pl.ds(off, _L)]
            o_v[pl.ds(off, _L)] = acc
            return _

        lax.fori_loop(0, chunk // _L, body, None)
        pltpu.sync_copy(o_v, out_hbm.at[b, pl.ds(base, chunk)])

    return k(m_pad, si_pad, mw_pad, ds_vec)


def kernel(mask_weight, boxes, scores, sigmas):
    bsz, _, n = boxes.shape
    npad = ((n + _T - 1) // _T) * _T
    n_masks = sigmas.shape[-1]

    si = jnp.argsort(-scores[:, 0], axis=1).astype(jnp.int32)
    si_pad = jnp.concatenate(
        [si, jnp.broadcast_to(jnp.arange(n, npad, dtype=jnp.int32),
                              (bsz, npad - n))], axis=1)
    boxes_pad = jnp.pad(boxes, ((0, 0), (0, 0), (0, npad - n)))

    coords, coords_t = _sc_sort_gather(boxes_pad, si_pad)
    coords_t = coords_t.reshape(bsz, npad, 8)
    m_pad = _run_passes(coords, coords_t, n)

    ds = -1.0 / (sigmas.reshape(n_masks) + 1e-12)
    ds_vec = jnp.broadcast_to(ds[:, None], (n_masks, _L))
    mw_pad = jnp.pad(mask_weight, ((0, 0), (0, 0), (0, npad - n)))
    out = _sc_unsort_combine(m_pad, si_pad, mw_pad, ds_vec)
    return out[:, None, :n]


# speedup vs baseline: 1.0248x; 1.0248x over previous
import functools

import jax
import jax.numpy as jnp
import numpy as np
from jax import lax
from jax.experimental import pallas as pl
from jax.experimental.pallas import tpu as pltpu
from jax.experimental.pallas import tpu_sc as plsc
from jax._src.pallas import primitives as _pl_primitives

_T = 1024
_BIG = 1e30
_NC, _NS, _L = 2, 16, 16
_NW = _NC * _NS


_CH = 8


def _pass1_tile(rowt_ref, col_ref, mask_ref, o2_ref, masked):
    col = col_ref[...]
    cx1 = col[:, 0:1, :]
    cy1 = col[:, 1:2, :]
    cx2 = col[:, 2:3, :]
    cy2 = col[:, 3:4, :]
    ca = col[:, 4:5, :]
    acc = None
    for ci in range(_T // _CH):
        r0 = ci * _CH
        rt = rowt_ref[:, r0:r0 + _CH, :]
        rx1 = rt[:, :, 0:1]
        ry1 = rt[:, :, 1:2]
        rx2 = rt[:, :, 2:3]
        ry2 = rt[:, :, 3:4]
        ra = rt[:, :, 4:5]
        iw = jnp.maximum(jnp.minimum(rx2, cx2) - jnp.maximum(rx1, cx1), 0.0)
        ih = jnp.maximum(jnp.minimum(ry2, cy2) - jnp.maximum(ry1, cy1), 0.0)
        inter = iw * ih
        union = ra + ca - inter
        iou = inter * _pl_primitives.reciprocal(union, approx=True)
        v = iou * iou
        if masked:
            v = v * mask_ref[r0:r0 + _CH, :][None]
        o2_ref[:, 0, r0:r0 + _CH, :] = v
        acc = v if acc is None else jnp.maximum(acc, v)
    return jnp.max(acc, axis=1, keepdims=True)


def _pass1_body(ik_ref, jk_ref, rowt_ref, col_ref, mask_ref,
                o_ref, o2_ref, ot_ref, omin_ref, *, n_real, bsz):
    k = pl.program_id(0)
    i = ik_ref[k]
    j = jk_ref[k]

    @pl.when(k == 0)
    def _init_min():
        omin_ref[...] = jnp.full_like(omin_ref, _BIG)

    @pl.when(i == 0)
    def _init():
        o_ref[...] = jnp.zeros_like(o_ref)

    def accum(masked):
        m = _pass1_tile(rowt_ref, col_ref, mask_ref, o2_ref, masked)
        o_ref[...] = jnp.maximum(o_ref[...],
                                 jnp.broadcast_to(m, o_ref.shape))

    @pl.when(i < j)
    def _upper():
        accum(False)

    @pl.when(i == j)
    def _diag():
        accum(True)
        mx2 = o_ref[:, 0:1, :]
        gj_lane = j * _T + lax.broadcasted_iota(jnp.int32, (1, 1, _T), 2)
        mx2 = jnp.where(gj_lane < n_real, mx2, _BIG)
        lane = lax.broadcasted_iota(jnp.int32, omin_ref.shape, 2)
        mn = jnp.min(mx2, axis=2, keepdims=True)
        omin_ref[...] = jnp.where(lane == j,
                                  jnp.broadcast_to(mn, omin_ref.shape),
                                  omin_ref[...])
        for bb in range(bsz):
            ot_ref[bb] = jnp.broadcast_to(mx2[bb], (8, _T)).T


def _pass2_body(ik_ref, jk_ref, t2_ref, mxt_ref, mn_ref, o_ref):
    k = pl.program_id(0)
    i = ik_ref[k]
    j = jk_ref[k]

    @pl.when(i == 0)
    def _init():
        lane = lax.broadcasted_iota(jnp.int32, mn_ref.shape, 2)
        s = jnp.min(jnp.where(lane > j, mn_ref[...], _BIG),
                    axis=(1, 2), keepdims=True)
        o_ref[...] = jnp.broadcast_to(-s, o_ref.shape)

    mx2c = mxt_ref[:, :, 0:1]
    acc = None
    for ci in range(_T // _CH):
        r0 = ci * _CH
        v = t2_ref[:, 0, r0:r0 + _CH, :] - mx2c[:, r0:r0 + _CH, :]
        acc = v if acc is None else jnp.maximum(acc, v)
    m = jnp.max(acc, axis=1, keepdims=True)
    o_ref[...] = jnp.maximum(o_ref[...], jnp.broadcast_to(m, o_ref.shape))


def _run_passes(coords, coords_t, n_real):
    bsz, _, npad = coords.shape
    nt = npad // _T
    pairs = [(i, j) for j in range(nt) for i in range(j + 1)]
    ik = jnp.asarray(np.array([p[0] for p in pairs], np.int32))
    jk = jnp.asarray(np.array([p[1] for p in pairs], np.int32))
    nk = len(pairs)
    cparams = pltpu.CompilerParams(
        dimension_semantics=("arbitrary",))

    mask = jnp.asarray(np.triu(np.ones((_T, _T), np.float32), 1))

    rowt_spec = pl.BlockSpec((bsz, _T, 8), lambda k_, ik_, jk_: (0, ik_[k_], 0))
    col_spec = pl.BlockSpec((bsz, 8, _T), lambda k_, ik_, jk_: (0, 0, jk_[k_]))
    mask_spec = pl.BlockSpec((_T, _T), lambda k_, ik_, jk_: (0, 0))
    out_spec = pl.BlockSpec((bsz, 8, _T), lambda k_, ik_, jk_: (0, 0, jk_[k_]))
    outt_spec = pl.BlockSpec((bsz, _T, 8), lambda k_, ik_, jk_: (0, jk_[k_], 0))
    min_spec = pl.BlockSpec((bsz, 8, 128), lambda k_, ik_, jk_: (0, 0, 0))
    tile_spec = pl.BlockSpec((bsz, 1, _T, _T), lambda k_, ik_, jk_: (0, k_, 0, 0))

    ioumax, iou2_tiles, mx2_t, tile_min2 = pl.pallas_call(
        functools.partial(_pass1_body, n_real=n_real, bsz=bsz),
        grid_spec=pltpu.PrefetchScalarGridSpec(
            num_scalar_prefetch=2,
            grid=(nk,),
            in_specs=[rowt_spec, col_spec, mask_spec],
            out_specs=[out_spec, tile_spec, outt_spec, min_spec],
        ),
        out_shape=[jax.ShapeDtypeStruct((bsz, 8, npad), jnp.float32),
                   jax.ShapeDtypeStruct((bsz, nk, _T, _T), jnp.float32),
                   jax.ShapeDtypeStruct((bsz, npad, 8), jnp.float32),
                   jax.ShapeDtypeStruct((bsz, 8, 128), jnp.float32)],
        compiler_params=cparams,
    )(ik, jk, coords_t, coords, mask)
    del ioumax

    m = pl.pallas_call(
        _pass2_body,
        grid_spec=pltpu.PrefetchScalarGridSpec(
            num_scalar_prefetch=2,
            grid=(nk,),
            in_specs=[tile_spec, rowt_spec, min_spec],
            out_specs=out_spec,
        ),
        out_shape=jax.ShapeDtypeStruct((bsz, 8, npad), jnp.float32),
        compiler_params=cparams,
    )(ik, jk, iou2_tiles, mx2_t, tile_min2)
    return m


def _sc_sort_gather(boxes_pad, si_pad):
    bsz, _, npad = boxes_pad.shape
    wpb = _NW // bsz
    chunk = npad // wpb
    nsteps = chunk // _L
    mesh = plsc.VectorSubcoreMesh(core_axis_name="c", subcore_axis_name="s")

    @functools.partial(
        pl.kernel, mesh=mesh,
        out_type=[jax.ShapeDtypeStruct((bsz, 8, npad), jnp.float32),
                  jax.ShapeDtypeStruct((bsz, npad * 8), jnp.float32)],
        scratch_types=[
            pltpu.VMEM((4 * npad,), jnp.float32),
            pltpu.VMEM((chunk,), jnp.int32),
            pltpu.VMEM((8, chunk), jnp.float32),
            pltpu.VMEM((chunk * 8,), jnp.float32),
        ],
        compiler_params=pltpu.CompilerParams(needs_layout_passes=False),
    )
    def k(boxes_hbm, si_hbm, out_hbm, outt_hbm, box_v, idx_v, o_v, ot_v):
        wid = lax.axis_index("s") * _NC + lax.axis_index("c")
        b = wid // wpb
        base = (wid % wpb) * chunk
        for c in range(4):
            pltpu.sync_copy(boxes_hbm.at[b, c],
                            box_v.at[pl.ds(c * npad, npad)])
        pltpu.sync_copy(si_hbm.at[b, pl.ds(base, chunk)], idx_v)
        zero = jnp.zeros((_L,), jnp.float32)
        lane = lax.iota(jnp.int32, _L)

        def body(s, _):
            off = s * _L
            idx = idx_v[pl.ds(off, _L)]
            cs = [plsc.load_gather(box_v, [idx + c * npad])
                  for c in range(4)]
            area = (jnp.maximum(cs[2] - cs[0], 0.0)
                    * jnp.maximum(cs[3] - cs[1], 0.0))
            tbase = (off + lane) * 8
            for c in range(4):
                o_v[c, pl.ds(off, _L)] = cs[c]
                plsc.store_scatter(ot_v, [tbase + c], cs[c])
            o_v[4, pl.ds(off, _L)] = area + 1e-12
            plsc.store_scatter(ot_v, [tbase + 4], area)
            for c in range(5, 8):
                o_v[c, pl.ds(off, _L)] = zero
            return _

        lax.fori_loop(0, nsteps, body, None)
        pltpu.sync_copy(o_v, out_hbm.at[b, pl.ds(0, 8), pl.ds(base, chunk)])
        pltpu.sync_copy(ot_v, outt_hbm.at[b, pl.ds(base * 8, chunk * 8)])

    return k(boxes_pad, si_pad)


def _sc_unsort_combine(m_pad, si_pad, mw_pad, ds_vec):
    bsz, _, npad = m_pad.shape
    n_masks = ds_vec.shape[0]
    wpb = _NW // bsz
    chunk = npad // wpb
    mesh = plsc.VectorSubcoreMesh(core_axis_name="c", subcore_axis_name="s")

    @functools.partial(
        pl.kernel, mesh=mesh,
        out_type=jax.ShapeDtypeStruct((bsz, npad), jnp.float32),
        scratch_types=[
            pltpu.VMEM((npad,), jnp.int32),
            pltpu.VMEM((npad,), jnp.int32),
            pltpu.VMEM((npad,), jnp.float32),
            pltpu.VMEM((n_masks, chunk), jnp.float32),
            pltpu.VMEM((n_masks, _L), jnp.float32),
            pltpu.VMEM((chunk,), jnp.float32),
        ],
        compiler_params=pltpu.CompilerParams(needs_layout_passes=False),
    )
    def k(m_hbm, si_hbm, mw_hbm, ds_hbm, out_hbm,
          si_v, un_v, m_v, w_v, ds_v, o_v):
        wid = lax.axis_index("s") * _NC + lax.axis_index("c")
        b = wid // wpb
        base = (wid % wpb) * chunk
        pltpu.sync_copy(si_hbm.at[b], si_v)
        pltpu.sync_copy(m_hbm.at[b, 0], m_v)
        pltpu.sync_copy(mw_hbm.at[b, pl.ds(0, n_masks), pl.ds(base, chunk)],
                        w_v)
        pltpu.sync_copy(ds_hbm, ds_v)
        lane = lax.iota(jnp.int32, _L)

        def scat(s, _):
            off = s * _L
            plsc.store_scatter(un_v, [si_v[pl.ds(off, _L)]], off + lane)
            return _

        lax.fori_loop(0, npad // _L, scat, None)

        def body(s, _):
            off = s * _L
            u = un_v[pl.ds(base + off, _L)]
            g = plsc.load_gather(m_v, [u])
            acc = jnp.zeros((_L,), jnp.float32)
            for mi in range(n_masks):
                acc = acc + jnp.exp(g * ds_v[mi]) * w_v[mi, pl.ds(off, _L)]
            o_v[pl.ds(off, _L)] = acc
            return _

        lax.fori_loop(0, chunk // _L, body, None)
        pltpu.sync_copy(o_v, out_hbm.at[b, pl.ds(base, chunk)])

    return k(m_pad, si_pad, mw_pad, ds_vec)


def kernel(mask_weight, boxes, scores, sigmas):
    bsz, _, n = boxes.shape
    npad = ((n + _T - 1) // _T) * _T
    n_masks = sigmas.shape[-1]

    si = jnp.argsort(-scores[:, 0], axis=1).astype(jnp.int32)
    si_pad = jnp.concatenate(
        [si, jnp.broadcast_to(jnp.arange(n, npad, dtype=jnp.int32),
                              (bsz, npad - n))], axis=1)
    boxes_pad = jnp.pad(boxes, ((0, 0), (0, 0), (0, npad - n)))

    coords, coords_t = _sc_sort_gather(boxes_pad, si_pad)
    coords_t = coords_t.reshape(bsz, npad, 8)
    m_pad = _run_passes(coords, coords_t, n)

    ds = -1.0 / (sigmas.reshape(n_masks) + 1e-12)
    ds_vec = jnp.broadcast_to(ds[:, None], (n_masks, _L))
    mw_pad = jnp.pad(mask_weight, ((0, 0), (0, 0), (0, npad - n)))
    out = _sc_unsort_combine(m_pad, si_pad, mw_pad, ds_vec)
    return out[:, None, :n]


# speedup vs baseline: 1.0329x; 1.0079x over previous
import functools

import jax
import jax.numpy as jnp
import numpy as np
from jax import lax
from jax.experimental import pallas as pl
from jax.experimental.pallas import tpu as pltpu
from jax.experimental.pallas import tpu_sc as plsc
from jax._src.pallas import primitives as _pl_primitives

_T = 1024
_BIG = 1e30
_NC, _NS, _L = 2, 16, 16
_NW = _NC * _NS


_CH = 8


def _pass1_tile(rowt_ref, col_ref, mask_ref, o2_ref, masked):
    col = col_ref[0]
    cx1 = col[0:1, :]
    cy1 = col[1:2, :]
    cx2 = col[2:3, :]
    cy2 = col[3:4, :]
    ca = col[4:5, :]
    acc = None
    for ci in range(_T // _CH):
        r0 = ci * _CH
        rt = rowt_ref[0, r0:r0 + _CH, :]
        rx1 = rt[:, 0:1]
        ry1 = rt[:, 1:2]
        rx2 = rt[:, 2:3]
        ry2 = rt[:, 3:4]
        ra = rt[:, 4:5]
        iw = jnp.maximum(jnp.minimum(rx2, cx2) - jnp.maximum(rx1, cx1), 0.0)
        ih = jnp.maximum(jnp.minimum(ry2, cy2) - jnp.maximum(ry1, cy1), 0.0)
        inter = iw * ih
        union = ra + ca - inter
        iou = inter * _pl_primitives.reciprocal(union, approx=True)
        v = iou * iou
        if masked:
            v = v * mask_ref[r0:r0 + _CH, :]
        o2_ref[0, 0, r0:r0 + _CH, :] = v
        acc = v if acc is None else jnp.maximum(acc, v)
    return jnp.max(acc, axis=0, keepdims=True)


def _pass1_body(ik_ref, jk_ref, rowt_ref, col_ref, mask_ref,
                o_ref, o2_ref, ot_ref, omin_ref, *, n_real):
    k = pl.program_id(1)
    i = ik_ref[k]
    j = jk_ref[k]

    @pl.when(k == 0)
    def _init_min():
        omin_ref[...] = jnp.full_like(omin_ref, _BIG)

    @pl.when(i == 0)
    def _init():
        o_ref[...] = jnp.zeros_like(o_ref)

    def accum(masked):
        m = _pass1_tile(rowt_ref, col_ref, mask_ref, o2_ref, masked)
        o_ref[0] = jnp.maximum(o_ref[0], jnp.broadcast_to(m, o_ref.shape[1:]))

    @pl.when(i < j)
    def _upper():
        accum(False)

    @pl.when(i == j)
    def _diag():
        accum(True)
        mx2 = o_ref[0, 0:1, :]
        gj_lane = j * _T + lax.broadcasted_iota(jnp.int32, (1, _T), 1)
        mx2 = jnp.where(gj_lane < n_real, mx2, _BIG)
        lane = lax.broadcasted_iota(jnp.int32, omin_ref.shape[1:], 1)
        omin_ref[0] = jnp.where(lane == j, jnp.min(mx2), omin_ref[0])
        ot_ref[0] = (jnp.broadcast_to(mx2, o_ref.shape[1:])).T


def _pass2_body(ik_ref, jk_ref, t2_ref, mxt_ref, mn_ref, o_ref):
    k = pl.program_id(1)
    i = ik_ref[k]
    j = jk_ref[k]

    @pl.when(i == 0)
    def _init():
        lane = lax.broadcasted_iota(jnp.int32, mn_ref.shape[1:], 1)
        s = -jnp.min(jnp.where(lane > j, mn_ref[0], _BIG))
        o_ref[...] = jnp.full_like(o_ref, 0.0) + s

    mx2c = mxt_ref[0][:, 0:1]
    acc = None
    for ci in range(_T // _CH):
        r0 = ci * _CH
        v = t2_ref[0, 0, r0:r0 + _CH, :] - mx2c[r0:r0 + _CH, :]
        acc = v if acc is None else jnp.maximum(acc, v)
    m = jnp.max(acc, axis=0, keepdims=True)
    o_ref[0] = jnp.maximum(o_ref[0], jnp.broadcast_to(m, o_ref.shape[1:]))


def _run_passes(coords, coords_t, n_real):
    b, _, npad = coords.shape
    nt = npad // _T
    pairs = [(i, j) for j in range(nt) for i in range(j + 1)]
    ik = jnp.asarray(np.array([p[0] for p in pairs], np.int32))
    jk = jnp.asarray(np.array([p[1] for p in pairs], np.int32))
    nk = len(pairs)
    cparams = pltpu.CompilerParams(
        dimension_semantics=("parallel", "arbitrary"))

    mask = jnp.asarray(np.triu(np.ones((_T, _T), np.float32), 1))

    rowt_spec = pl.BlockSpec((1, _T, 8), lambda b_, k_, ik_, jk_: (b_, ik_[k_], 0))
    col_spec = pl.BlockSpec((1, 8, _T), lambda b_, k_, ik_, jk_: (b_, 0, jk_[k_]))
    mask_spec = pl.BlockSpec((_T, _T), lambda b_, k_, ik_, jk_: (0, 0))
    out_spec = pl.BlockSpec((1, 8, _T), lambda b_, k_, ik_, jk_: (b_, 0, jk_[k_]))
    outt_spec = pl.BlockSpec((1, _T, 8), lambda b_, k_, ik_, jk_: (b_, jk_[k_], 0))
    min_spec = pl.BlockSpec((1, 8, 128), lambda b_, k_, ik_, jk_: (b_, 0, 0))

    tile_spec = pl.BlockSpec((1, 1, _T, _T), lambda b_, k_, ik_, jk_: (b_, k_, 0, 0))

    ioumax, iou2_tiles, mx2_t, tile_min2 = pl.pallas_call(
        functools.partial(_pass1_body, n_real=n_real),
        grid_spec=pltpu.PrefetchScalarGridSpec(
            num_scalar_prefetch=2,
            grid=(b, nk),
            in_specs=[rowt_spec, col_spec, mask_spec],
            out_specs=[out_spec, tile_spec, outt_spec, min_spec],
        ),
        out_shape=[jax.ShapeDtypeStruct((b, 8, npad), jnp.float32),
                   jax.ShapeDtypeStruct((b, nk, _T, _T), jnp.float32),
                   jax.ShapeDtypeStruct((b, npad, 8), jnp.float32),
                   jax.ShapeDtypeStruct((b, 8, 128), jnp.float32)],
        compiler_params=cparams,
    )(ik, jk, coords_t, coords, mask)
    del ioumax

    m = pl.pallas_call(
        _pass2_body,
        grid_spec=pltpu.PrefetchScalarGridSpec(
            num_scalar_prefetch=2,
            grid=(b, nk),
            in_specs=[tile_spec, rowt_spec, min_spec],
            out_specs=out_spec,
        ),
        out_shape=jax.ShapeDtypeStruct((b, 8, npad), jnp.float32),
        compiler_params=cparams,
    )(ik, jk, iou2_tiles, mx2_t, tile_min2)
    return m


def _sc_sort_gather(boxes_pad, si_pad):
    bsz, _, npad = boxes_pad.shape
    wpb = _NW // bsz
    chunk = npad // wpb
    nsteps = chunk // _L
    mesh = plsc.VectorSubcoreMesh(core_axis_name="c", subcore_axis_name="s")

    @functools.partial(
        pl.kernel, mesh=mesh,
        out_type=[jax.ShapeDtypeStruct((bsz, 8, npad), jnp.float32),
                  jax.ShapeDtypeStruct((bsz, npad * 8), jnp.float32)],
        scratch_types=[
            pltpu.VMEM((4 * npad,), jnp.float32),
            pltpu.VMEM((chunk,), jnp.int32),
            pltpu.VMEM((8, chunk), jnp.float32),
            pltpu.VMEM((chunk * 8,), jnp.float32),
        ],
        compiler_params=pltpu.CompilerParams(needs_layout_passes=False),
    )
    def k(boxes_hbm, si_hbm, out_hbm, outt_hbm, box_v, idx_v, o_v, ot_v):
        wid = lax.axis_index("s") * _NC + lax.axis_index("c")
        b = wid // wpb
        base = (wid % wpb) * chunk
        for c in range(4):
            pltpu.sync_copy(boxes_hbm.at[b, c],
                            box_v.at[pl.ds(c * npad, npad)])
        pltpu.sync_copy(si_hbm.at[b, pl.ds(base, chunk)], idx_v)
        zero = jnp.zeros((_L,), jnp.float32)
        lane = lax.iota(jnp.int32, _L)

        def body(s, _):
            off = s * _L
            idx = idx_v[pl.ds(off, _L)]
            cs = [plsc.load_gather(box_v, [idx + c * npad])
                  for c in range(4)]
            area = (jnp.maximum(cs[2] - cs[0], 0.0)
                    * jnp.maximum(cs[3] - cs[1], 0.0))
            tbase = (off + lane) * 8
            for c in range(4):
                o_v[c, pl.ds(off, _L)] = cs[c]
                plsc.store_scatter(ot_v, [tbase + c], cs[c])
            o_v[4, pl.ds(off, _L)] = area + 1e-12
            plsc.store_scatter(ot_v, [tbase + 4], area)
            for c in range(5, 8):
                o_v[c, pl.ds(off, _L)] = zero
            return _

        lax.fori_loop(0, nsteps, body, None)
        pltpu.sync_copy(o_v, out_hbm.at[b, pl.ds(0, 8), pl.ds(base, chunk)])
        pltpu.sync_copy(ot_v, outt_hbm.at[b, pl.ds(base * 8, chunk * 8)])

    return k(boxes_pad, si_pad)


def _sc_unsort_combine(m_pad, si_pad, mw_pad, ds_vec):
    bsz, _, npad = m_pad.shape
    n_masks = ds_vec.shape[0]
    wpb = _NW // bsz
    chunk = npad // wpb
    mesh = plsc.VectorSubcoreMesh(core_axis_name="c", subcore_axis_name="s")

    @functools.partial(
        pl.kernel, mesh=mesh,
        out_type=jax.ShapeDtypeStruct((bsz, npad), jnp.float32),
        scratch_types=[
            pltpu.VMEM((npad,), jnp.int32),
            pltpu.VMEM((npad,), jnp.int32),
            pltpu.VMEM((npad,), jnp.float32),
            pltpu.VMEM((n_masks, chunk), jnp.float32),
            pltpu.VMEM((n_masks, _L), jnp.float32),
            pltpu.VMEM((chunk,), jnp.float32),
        ],
        compiler_params=pltpu.CompilerParams(needs_layout_passes=False),
    )
    def k(m_hbm, si_hbm, mw_hbm, ds_hbm, out_hbm,
          si_v, un_v, m_v, w_v, ds_v, o_v):
        wid = lax.axis_index("s") * _NC + lax.axis_index("c")
        b = wid // wpb
        base = (wid % wpb) * chunk
        pltpu.sync_copy(si_hbm.at[b], si_v)
        pltpu.sync_copy(m_hbm.at[b, 0], m_v)
        pltpu.sync_copy(mw_hbm.at[b, pl.ds(0, n_masks), pl.ds(base, chunk)],
                        w_v)
        pltpu.sync_copy(ds_hbm, ds_v)
        lane = lax.iota(jnp.int32, _L)

        def scat(s, _):
            off = s * _L
            plsc.store_scatter(un_v, [si_v[pl.ds(off, _L)]], off + lane)
            return _

        lax.fori_loop(0, npad // _L, scat, None)

        def body(s, _):
            off = s * _L
            u = un_v[pl.ds(base + off, _L)]
            g = plsc.load_gather(m_v, [u])
            acc = jnp.zeros((_L,), jnp.float32)
            for mi in range(n_masks):
                acc = acc + jnp.exp(g * ds_v[mi]) * w_v[mi, pl.ds(off, _L)]
            o_v[pl.ds(off, _L)] = acc
            return _

        lax.fori_loop(0, chunk // _L, body, None)
        pltpu.sync_copy(o_v, out_hbm.at[b, pl.ds(base, chunk)])

    return k(m_pad, si_pad, mw_pad, ds_vec)


def kernel(mask_weight, boxes, scores, sigmas):
    bsz, _, n = boxes.shape
    npad = ((n + _T - 1) // _T) * _T
    n_masks = sigmas.shape[-1]

    si = jnp.argsort(-scores[:, 0], axis=1).astype(jnp.int32)
    si_pad = jnp.concatenate(
        [si, jnp.broadcast_to(jnp.arange(n, npad, dtype=jnp.int32),
                              (bsz, npad - n))], axis=1)
    boxes_pad = jnp.pad(boxes, ((0, 0), (0, 0), (0, npad - n)))

    coords, coords_t = _sc_sort_gather(boxes_pad, si_pad)
    coords_t = coords_t.reshape(bsz, npad, 8)
    m_pad = _run_passes(coords, coords_t, n)

    ds = -1.0 / (sigmas.reshape(n_masks) + 1e-12)
    ds_vec = jnp.broadcast_to(ds[:, None], (n_masks, _L))
    mw_pad = jnp.pad(mask_weight, ((0, 0), (0, 0), (0, npad - n)))
    out = _sc_unsort_combine(m_pad, si_pad, mw_pad, ds_vec)
    return out[:, None, :n]
